# baseline (speedup 1.0000x reference)
import functools

import jax
import jax.numpy as jnp
from jax import lax
from jax.experimental import pallas as pl
from jax.experimental.pallas import tpu as pltpu
from jax.experimental.pallas import tpu_sc as plsc

D_H = 1024
D_E = 512
N_E = 16
T_TOTAL = 4096
N_PAIR = 2 * T_TOTAL
TBLK = 512
NT = T_TOTAL // TBLK

BLK = 512
N_PAD = N_PAIR + N_E * BLK
NBLK = N_PAD // BLK

NC, NS = 2, 16
NW = NC * NS
TOK_W = T_TOTAL // NW
PAIR_W = 2 * TOK_W


def _silu(v):
    return v * jax.nn.sigmoid(v)


def _dot_t(a, b):
    return lax.dot_general(a, b, (((1,), (1,)), ((), ())),
                           preferred_element_type=jnp.float32)


def _k0_body(x_ref, wgr_ref, epair_ref, wpair_ref, hist_ref):
    xb = x_ref[...]
    logits = _dot_t(xb, wgr_ref[...])
    m = jnp.max(logits, axis=1, keepdims=True)
    p = jnp.exp(logits - m)
    p = p / jnp.sum(p, axis=1, keepdims=True)
    iota = lax.broadcasted_iota(jnp.int32, p.shape, 1)
    m1 = jnp.max(p, axis=1, keepdims=True)
    e1 = jnp.min(jnp.where(p == m1, iota, N_E), axis=1, keepdims=True)
    pm = jnp.where(iota == e1, -jnp.inf, p)
    m2 = jnp.max(pm, axis=1, keepdims=True)
    e2 = jnp.min(jnp.where(pm == m2, iota, N_E), axis=1, keepdims=True)
    epair_ref[...] = jnp.concatenate([e1, e2], axis=1)
    wpair_ref[...] = jnp.concatenate([m1, m2], axis=1)
    oh = ((iota == e1) | (iota == e2)).astype(jnp.int32)
    ri = lax.broadcasted_iota(jnp.int32, (TBLK, N_E), 0)
    for cc in range(TBLK // TOK_W):
        msk = (ri >= cc * TOK_W) & (ri < (cc + 1) * TOK_W)
        r = jnp.sum(jnp.where(msk, oh, 0), axis=0)
        hist_ref[0, pl.ds(cc, 1), :] = r[None, :]


def _k0(x_flat, Wg_router):
    return pl.pallas_call(
        _k0_body,
        grid=(NT,),
        in_specs=[
            pl.BlockSpec((TBLK, D_H), lambda i: (i, 0)),
            pl.BlockSpec((N_E, D_H), lambda i: (0, 0)),
        ],
        out_specs=[
            pl.BlockSpec((TBLK, 2), lambda i: (i, 0)),
            pl.BlockSpec((TBLK, 2), lambda i: (i, 0)),
            pl.BlockSpec((1, TBLK // TOK_W, N_E), lambda i: (i, 0, 0)),
        ],
        out_shape=[
            jax.ShapeDtypeStruct((T_TOTAL, 2), jnp.int32),
            jax.ShapeDtypeStruct((T_TOTAL, 2), jnp.float32),
            jax.ShapeDtypeStruct((NT, TBLK // TOK_W, N_E), jnp.int32),
        ],
    )(x_flat, Wg_router)


def _k1s_body(x_ref, wsg_ref, wsu_ref, wsd_ref, shared_ref):
    xb = x_ref[...].astype(jnp.bfloat16)
    sg = _dot_t(xb, wsg_ref[...].astype(jnp.bfloat16))
    su = _dot_t(xb, wsu_ref[...].astype(jnp.bfloat16))
    h = (_silu(sg) * su).astype(jnp.bfloat16)
    shared_ref[...] = _dot_t(h, wsd_ref[...].astype(jnp.bfloat16))


def _k1s(x_flat, Ws_gate, Ws_up, Ws_down):
    return pl.pallas_call(
        _k1s_body,
        grid=(NT,),
        in_specs=[
            pl.BlockSpec((TBLK, D_H), lambda i: (i, 0)),
            pl.BlockSpec((D_H, D_H), lambda i: (0, 0)),
            pl.BlockSpec((D_H, D_H), lambda i: (0, 0)),
            pl.BlockSpec((D_H, D_H), lambda i: (0, 0)),
        ],
        out_specs=pl.BlockSpec((TBLK, D_H), lambda i: (i, 0)),
        out_shape=jax.ShapeDtypeStruct((T_TOTAL, D_H), jnp.float32),
    )(x_flat, Ws_gate, Ws_up, Ws_down)


def _k2(epair_flat, x_flat, hist):
    mesh = plsc.VectorSubcoreMesh(core_axis_name="c", subcore_axis_name="s",
                                  num_cores=NC, num_subcores=NS)

    @functools.partial(
        pl.kernel,
        out_type=(
            jax.ShapeDtypeStruct((N_PAIR,), jnp.int32),
            jax.ShapeDtypeStruct((N_PAD, D_H), jnp.float32),
            jax.ShapeDtypeStruct((NBLK,), jnp.int32),
            jax.ShapeDtypeStruct((NBLK,), jnp.int32),
            jax.ShapeDtypeStruct((NBLK,), jnp.int32),
        ),
        mesh=mesh,
        scratch_types=[
            pltpu.VMEM((PAIR_W,), jnp.int32),
            pltpu.VMEM((PAIR_W,), jnp.int32),
            pltpu.VMEM((NW, 16), jnp.int32),
            pltpu.VMEM((64, D_H), jnp.float32),
            pltpu.VMEM((64,), jnp.int32),
            pltpu.VMEM((64,), jnp.int32),
            pltpu.VMEM((NBLK,), jnp.int32),
            pltpu.VMEM((NBLK,), jnp.int32),
            pltpu.VMEM((NBLK,), jnp.int32),
            pltpu.SemaphoreType.DMA,
            pltpu.SemaphoreType.DMA,
        ],
        compiler_params=pltpu.CompilerParams(needs_layout_passes=False),
    )
    def k2(epair_hbm, x_hbm, hist_hbm,
           pos_hbm, xs_hbm, bmap_hbm, xsblk_hbm, bvalid_hbm,
           ids_a, pos_v, hist_all, rows_v,
           idx_a, idx_b, bstage, xstage, vstage,
           sem_a, sem_b):
        c = lax.axis_index("c")
        s = lax.axis_index("s")
        w_own = c * NS + s
        iota16 = lax.iota(jnp.int32, 16)
        zeros16 = jnp.zeros((16,), jnp.int32)

        pltpu.sync_copy(epair_hbm.at[pl.ds(w_own * PAIR_W, PAIR_W)], ids_a)
        pltpu.sync_copy(hist_hbm, hist_all)

        def acc_body(w, carry):
            counts, before = carry
            row = hist_all[w]
            take = jnp.full((16,), w) < jnp.full((16,), w_own)
            return (counts + row,
                    before + jnp.where(take, row, zeros16))

        counts, before = lax.fori_loop(0, NW, acc_body, (zeros16, zeros16))

        nblk = (counts + (BLK - 1)) // BLK
        cum_incl = plsc.cumsum(nblk)
        off = (cum_incl - nblk) * BLK
        slot_base = off + before

        def rank_body(j, ctr):
            v = ids_a[pl.ds(j * 16, 16)]
            base = ctr.at[v].get(mode="promise_in_bounds")
            within = zeros16
            for e in range(N_E):
                mask = v == e
                mi = mask.astype(jnp.int32)
                cs = plsc.cumsum(mi)
                within = jnp.where(mask, cs - 1, within)
                tot = jnp.sum(mi)
                ctr = jnp.where(iota16 == e, ctr + tot, ctr)
            pos_v[pl.ds(j * 16, 16)] = base + within
            return ctr

        lax.fori_loop(0, PAIR_W // 16, rank_body, slot_base)

        pltpu.sync_copy(pos_v, pos_hbm.at[pl.ds(w_own * PAIR_W, PAIR_W)])

        tok0 = w_own * TOK_W
        for ch in range(2):
            pltpu.sync_copy(x_hbm.at[pl.ds(tok0 + ch * 64, 64)], rows_v)
            for i in range(4):
                g0 = jnp.full((16,), ch * 128) + 2 * (i * 16 + iota16)
                idx_a[pl.ds(i * 16, 16)] = plsc.load_gather(pos_v, [g0])
                idx_b[pl.ds(i * 16, 16)] = plsc.load_gather(pos_v, [g0 + 1])
            cp_a = pltpu.async_copy(rows_v, xs_hbm.at[idx_a], sem_a)
            cp_b = pltpu.async_copy(rows_v, xs_hbm.at[idx_b], sem_b)
            cp_a.wait()
            cp_b.wait()

        @pl.when(jnp.logical_and(c == 0, s == 0))
        def _():
            used = jnp.sum(nblk)
            for j in range(NBLK // 16):
                bb = iota16 + j * 16
                ee = zeros16
                for e in range(N_E):
                    ci = jnp.sum(jnp.where(iota16 == e, cum_incl, zeros16))
                    ee = ee + (bb >= jnp.full((16,), ci)).astype(jnp.int32)
                ee = jnp.minimum(ee, N_E - 1)
                bstage[pl.ds(j * 16, 16)] = ee
                xstage[pl.ds(j * 16, 16)] = jnp.minimum(
                    bb, jnp.full((16,), used - 1))
                vstage[pl.ds(j * 16, 16)] = (
                    bb < jnp.full((16,), used)).astype(jnp.int32)
            pltpu.sync_copy(bstage, bmap_hbm)
            pltpu.sync_copy(xstage, xsblk_hbm)
            pltpu.sync_copy(vstage, bvalid_hbm)

    return k2(epair_flat, x_flat, hist)


def _k3_body(xsblk_ref, bmap_ref, bvalid_ref,
             xs_ref, wg_ref, wu_ref, wd_ref, ys_ref):
    b = pl.program_id(0)

    @pl.when(bvalid_ref[b] == 1)
    def _():
        xb = xs_ref[...].astype(jnp.bfloat16)
        g = _dot_t(xb, wg_ref[0].astype(jnp.bfloat16))
        u = _dot_t(xb, wu_ref[0].astype(jnp.bfloat16))
        h = (_silu(g) * u).astype(jnp.bfloat16)
        ys_ref[...] = _dot_t(h, wd_ref[0].astype(jnp.bfloat16))


def _k3(xs, W_gate, W_up, W_down, xsblk, bmap, bvalid):
    grid_spec = pltpu.PrefetchScalarGridSpec(
        num_scalar_prefetch=3,
        grid=(NBLK,),
        in_specs=[
            pl.BlockSpec((BLK, D_H), lambda b, xi, bm, bv: (xi[b], 0)),
            pl.BlockSpec((1, D_E, D_H), lambda b, xi, bm, bv: (bm[b], 0, 0)),
            pl.BlockSpec((1, D_E, D_H), lambda b, xi, bm, bv: (bm[b], 0, 0)),
            pl.BlockSpec((1, D_H, D_E), lambda b, xi, bm, bv: (bm[b], 0, 0)),
        ],
        out_specs=pl.BlockSpec((BLK, D_H), lambda b, xi, bm, bv: (xi[b], 0)),
    )
    return pl.pallas_call(
        _k3_body,
        grid_spec=grid_spec,
        out_shape=jax.ShapeDtypeStruct((N_PAD, D_H), jnp.float32),
    )(xsblk, bmap, bvalid, xs, W_gate, W_up, W_down)


def _k4sc(ys, pos):
    mesh = plsc.VectorSubcoreMesh(core_axis_name="c", subcore_axis_name="s",
                                  num_cores=NC, num_subcores=NS)

    @functools.partial(
        pl.kernel,
        out_type=jax.ShapeDtypeStruct((N_PAIR, D_H), jnp.float32),
        mesh=mesh,
        scratch_types=[
            pltpu.VMEM((PAIR_W,), jnp.int32),
            pltpu.VMEM((32, D_H), jnp.float32),
            pltpu.VMEM((32, D_H), jnp.float32),
            pltpu.VMEM((32,), jnp.int32),
            pltpu.VMEM((32,), jnp.int32),
            pltpu.SemaphoreType.DMA,
            pltpu.SemaphoreType.DMA,
            pltpu.SemaphoreType.DMA,
            pltpu.SemaphoreType.DMA,
        ],
        compiler_params=pltpu.CompilerParams(needs_layout_passes=False),
    )
    def k4(ys_hbm, pos_hbm, yp_hbm, posall, yr0, yr1, ix0, ix1,
           sg0, sg1, sw0, sw1):
        c = lax.axis_index("c")
        s = lax.axis_index("s")
        w_own = c * NS + s
        pltpu.sync_copy(pos_hbm.at[pl.ds(w_own * PAIR_W, PAIR_W)], posall)
        bufs = (yr0, yr1)
        idxs = (ix0, ix1)
        gsems = (sg0, sg1)
        wsems = (sw0, sw1)
        writes = [None] * 8
        for ch in range(8):
            b = ch % 2
            if ch >= 2:
                writes[ch - 2].wait()
            ii = idxs[b]
            ii[pl.ds(0, 16)] = posall[pl.ds(ch * 32, 16)]
            ii[pl.ds(16, 16)] = posall[pl.ds(ch * 32 + 16, 16)]
            pltpu.async_copy(ys_hbm.at[ii], bufs[b], gsems[b]).wait()
            writes[ch] = pltpu.async_copy(
                bufs[b], yp_hbm.at[pl.ds(w_own * PAIR_W + ch * 32, 32)],
                wsems[b])
        writes[6].wait()
        writes[7].wait()

    return k4(ys, pos)


def _k5_body(shared_ref, yp_ref, wp_ref, out_ref):
    w0 = wp_ref[...][:, 0:1]
    w1 = wp_ref[...][:, 1:2]
    yp = yp_ref[...]
    out_ref[...] = (shared_ref[...] + w0 * yp[:, :D_H] + w1 * yp[:, D_H:])


def _k5(shared, ypairs2, wpair):
    return pl.pallas_call(
        _k5_body,
        grid=(NT,),
        in_specs=[
            pl.BlockSpec((TBLK, D_H), lambda i: (i, 0)),
            pl.BlockSpec((TBLK, 2 * D_H), lambda i: (i, 0)),
            pl.BlockSpec((TBLK, 2), lambda i: (i, 0)),
        ],
        out_specs=pl.BlockSpec((TBLK, D_H), lambda i: (i, 0)),
        out_shape=jax.ShapeDtypeStruct((T_TOTAL, D_H), jnp.float32),
    )(shared, ypairs2, wpair)


def kernel(x, Wg_router, W_gate, W_up, W_down, Ws_gate, Ws_up, Ws_down):
    B, S, H = x.shape
    x_flat = x.reshape(-1, H)
    epair, wpair, hist = _k0(x_flat, Wg_router)
    pos, xs, bmap, xsblk, bvalid = _k2(epair.reshape(-1), x_flat,
                                       hist.reshape(NW, N_E))
    ys = _k3(xs, W_gate, W_up, W_down, xsblk, bmap, bvalid)
    ypairs = _k4sc(ys, pos)
    shared = _k1s(x_flat, Ws_gate, Ws_up, Ws_down)
    out = _k5(shared, ypairs.reshape(T_TOTAL, 2 * D_H), wpair)
    return out.reshape(B, S, H)


# speedup vs baseline: 1.0061x; 1.0061x over previous
import functools

import jax
import jax.numpy as jnp
from jax import lax
from jax.experimental import pallas as pl
from jax.experimental.pallas import tpu as pltpu
from jax.experimental.pallas import tpu_sc as plsc

D_H = 1024
D_E = 512
N_E = 16
T_TOTAL = 4096
N_PAIR = 2 * T_TOTAL
TBLK = 512
NT = T_TOTAL // TBLK

BLK = 512
N_PAD = N_PAIR + N_E * BLK
NBLK = N_PAD // BLK

NC, NS = 2, 16
NW = NC * NS
TOK_W = T_TOTAL // NW
PAIR_W = 2 * TOK_W


def _silu(v):
    return v * jax.nn.sigmoid(v)


def _dot_t(a, b):
    return lax.dot_general(a, b, (((1,), (1,)), ((), ())),
                           preferred_element_type=jnp.float32)


def _k0_body(x_ref, wgr_ref, epair_ref, wpair_ref, hist_ref):
    xb = x_ref[...]
    logits = _dot_t(xb, wgr_ref[...])
    m = jnp.max(logits, axis=1, keepdims=True)
    p = jnp.exp(logits - m)
    p = p / jnp.sum(p, axis=1, keepdims=True)
    iota = lax.broadcasted_iota(jnp.int32, p.shape, 1)
    m1 = jnp.max(p, axis=1, keepdims=True)
    e1 = jnp.min(jnp.where(p == m1, iota, N_E), axis=1, keepdims=True)
    pm = jnp.where(iota == e1, -jnp.inf, p)
    m2 = jnp.max(pm, axis=1, keepdims=True)
    e2 = jnp.min(jnp.where(pm == m2, iota, N_E), axis=1, keepdims=True)
    epair_ref[...] = jnp.concatenate([e1, e2], axis=1)
    wpair_ref[...] = jnp.concatenate([m1, m2], axis=1)
    oh = ((iota == e1) | (iota == e2)).astype(jnp.int32)
    ri = lax.broadcasted_iota(jnp.int32, (TBLK, N_E), 0)
    for cc in range(TBLK // TOK_W):
        msk = (ri >= cc * TOK_W) & (ri < (cc + 1) * TOK_W)
        r = jnp.sum(jnp.where(msk, oh, 0), axis=0)
        hist_ref[0, pl.ds(cc, 1), :] = r[None, :]


def _k0(x_flat, Wg_router):
    return pl.pallas_call(
        _k0_body,
        grid=(NT,),
        in_specs=[
            pl.BlockSpec((TBLK, D_H), lambda i: (i, 0)),
            pl.BlockSpec((N_E, D_H), lambda i: (0, 0)),
        ],
        out_specs=[
            pl.BlockSpec((TBLK, 2), lambda i: (i, 0)),
            pl.BlockSpec((TBLK, 2), lambda i: (i, 0)),
            pl.BlockSpec((1, TBLK // TOK_W, N_E), lambda i: (i, 0, 0)),
        ],
        out_shape=[
            jax.ShapeDtypeStruct((T_TOTAL, 2), jnp.int32),
            jax.ShapeDtypeStruct((T_TOTAL, 2), jnp.float32),
            jax.ShapeDtypeStruct((NT, TBLK // TOK_W, N_E), jnp.int32),
        ],
    )(x_flat, Wg_router)


def _k1s_body(x_ref, wsg_ref, wsu_ref, wsd_ref, shared_ref):
    xb = x_ref[...]
    sg = _dot_t(xb, wsg_ref[...])
    su = _dot_t(xb, wsu_ref[...])
    h = _silu(sg) * su
    shared_ref[...] = _dot_t(h, wsd_ref[...])


def _k1s(x_flat, Ws_gate, Ws_up, Ws_down):
    return pl.pallas_call(
        _k1s_body,
        grid=(NT,),
        in_specs=[
            pl.BlockSpec((TBLK, D_H), lambda i: (i, 0)),
            pl.BlockSpec((D_H, D_H), lambda i: (0, 0)),
            pl.BlockSpec((D_H, D_H), lambda i: (0, 0)),
            pl.BlockSpec((D_H, D_H), lambda i: (0, 0)),
        ],
        out_specs=pl.BlockSpec((TBLK, D_H), lambda i: (i, 0)),
        out_shape=jax.ShapeDtypeStruct((T_TOTAL, D_H), jnp.float32),
    )(x_flat, Ws_gate, Ws_up, Ws_down)


def _k2(epair_flat, x_flat, hist):
    mesh = plsc.VectorSubcoreMesh(core_axis_name="c", subcore_axis_name="s",
                                  num_cores=NC, num_subcores=NS)

    @functools.partial(
        pl.kernel,
        out_type=(
            jax.ShapeDtypeStruct((N_PAIR,), jnp.int32),
            jax.ShapeDtypeStruct((N_PAD, D_H), jnp.float32),
            jax.ShapeDtypeStruct((NBLK,), jnp.int32),
            jax.ShapeDtypeStruct((NBLK,), jnp.int32),
            jax.ShapeDtypeStruct((NBLK,), jnp.int32),
        ),
        mesh=mesh,
        scratch_types=[
            pltpu.VMEM((PAIR_W,), jnp.int32),
            pltpu.VMEM((PAIR_W,), jnp.int32),
            pltpu.VMEM((NW, 16), jnp.int32),
            pltpu.VMEM((64, D_H), jnp.float32),
            pltpu.VMEM((64,), jnp.int32),
            pltpu.VMEM((64,), jnp.int32),
            pltpu.VMEM((NBLK,), jnp.int32),
            pltpu.VMEM((NBLK,), jnp.int32),
            pltpu.VMEM((NBLK,), jnp.int32),
            pltpu.SemaphoreType.DMA,
            pltpu.SemaphoreType.DMA,
        ],
        compiler_params=pltpu.CompilerParams(needs_layout_passes=False),
    )
    def k2(epair_hbm, x_hbm, hist_hbm,
           pos_hbm, xs_hbm, bmap_hbm, xsblk_hbm, bvalid_hbm,
           ids_a, pos_v, hist_all, rows_v,
           idx_a, idx_b, bstage, xstage, vstage,
           sem_a, sem_b):
        c = lax.axis_index("c")
        s = lax.axis_index("s")
        w_own = c * NS + s
        iota16 = lax.iota(jnp.int32, 16)
        zeros16 = jnp.zeros((16,), jnp.int32)

        pltpu.sync_copy(epair_hbm.at[pl.ds(w_own * PAIR_W, PAIR_W)], ids_a)
        pltpu.sync_copy(hist_hbm, hist_all)

        def acc_body(w, carry):
            counts, before = carry
            row = hist_all[w]
            take = jnp.full((16,), w) < jnp.full((16,), w_own)
            return (counts + row,
                    before + jnp.where(take, row, zeros16))

        counts, before = lax.fori_loop(0, NW, acc_body, (zeros16, zeros16))

        nblk = (counts + (BLK - 1)) // BLK
        cum_incl = plsc.cumsum(nblk)
        off = (cum_incl - nblk) * BLK
        slot_base = off + before

        def rank_body(j, ctr):
            v = ids_a[pl.ds(j * 16, 16)]
            base = ctr.at[v].get(mode="promise_in_bounds")
            within = zeros16
            for e in range(N_E):
                mask = v == e
                mi = mask.astype(jnp.int32)
                cs = plsc.cumsum(mi)
                within = jnp.where(mask, cs - 1, within)
                tot = jnp.sum(mi)
                ctr = jnp.where(iota16 == e, ctr + tot, ctr)
            pos_v[pl.ds(j * 16, 16)] = base + within
            return ctr

        lax.fori_loop(0, PAIR_W // 16, rank_body, slot_base)

        pltpu.sync_copy(pos_v, pos_hbm.at[pl.ds(w_own * PAIR_W, PAIR_W)])

        tok0 = w_own * TOK_W
        for ch in range(2):
            pltpu.sync_copy(x_hbm.at[pl.ds(tok0 + ch * 64, 64)], rows_v)
            for i in range(4):
                g0 = jnp.full((16,), ch * 128) + 2 * (i * 16 + iota16)
                idx_a[pl.ds(i * 16, 16)] = plsc.load_gather(pos_v, [g0])
                idx_b[pl.ds(i * 16, 16)] = plsc.load_gather(pos_v, [g0 + 1])
            cp_a = pltpu.async_copy(rows_v, xs_hbm.at[idx_a], sem_a)
            cp_b = pltpu.async_copy(rows_v, xs_hbm.at[idx_b], sem_b)
            cp_a.wait()
            cp_b.wait()

        @pl.when(jnp.logical_and(c == 0, s == 0))
        def _():
            used = jnp.sum(nblk)
            for j in range(NBLK // 16):
                bb = iota16 + j * 16
                ee = zeros16
                for e in range(N_E):
                    ci = jnp.sum(jnp.where(iota16 == e, cum_incl, zeros16))
                    ee = ee + (bb >= jnp.full((16,), ci)).astype(jnp.int32)
                ee = jnp.minimum(ee, N_E - 1)
                bstage[pl.ds(j * 16, 16)] = ee
                xstage[pl.ds(j * 16, 16)] = jnp.minimum(
                    bb, jnp.full((16,), used - 1))
                vstage[pl.ds(j * 16, 16)] = (
                    bb < jnp.full((16,), used)).astype(jnp.int32)
            pltpu.sync_copy(bstage, bmap_hbm)
            pltpu.sync_copy(xstage, xsblk_hbm)
            pltpu.sync_copy(vstage, bvalid_hbm)

    return k2(epair_flat, x_flat, hist)


def _k3_body(xsblk_ref, bmap_ref, bvalid_ref,
             xs_ref, wg_ref, wu_ref, wd_ref, ys_ref):
    b = pl.program_id(0)

    @pl.when(bvalid_ref[b] == 1)
    def _():
        xb = xs_ref[...]
        g = _dot_t(xb, wg_ref[0])
        u = _dot_t(xb, wu_ref[0])
        h = _silu(g) * u
        ys_ref[...] = _dot_t(h, wd_ref[0])


def _k3(xs, W_gate, W_up, W_down, xsblk, bmap, bvalid):
    grid_spec = pltpu.PrefetchScalarGridSpec(
        num_scalar_prefetch=3,
        grid=(NBLK,),
        in_specs=[
            pl.BlockSpec((BLK, D_H), lambda b, xi, bm, bv: (xi[b], 0)),
            pl.BlockSpec((1, D_E, D_H), lambda b, xi, bm, bv: (bm[b], 0, 0)),
            pl.BlockSpec((1, D_E, D_H), lambda b, xi, bm, bv: (bm[b], 0, 0)),
            pl.BlockSpec((1, D_H, D_E), lambda b, xi, bm, bv: (bm[b], 0, 0)),
        ],
        out_specs=pl.BlockSpec((BLK, D_H), lambda b, xi, bm, bv: (xi[b], 0)),
    )
    return pl.pallas_call(
        _k3_body,
        grid_spec=grid_spec,
        out_shape=jax.ShapeDtypeStruct((N_PAD, D_H), jnp.float32),
    )(xsblk, bmap, bvalid, xs, W_gate, W_up, W_down)


def _k4sc(ys, pos):
    mesh = plsc.VectorSubcoreMesh(core_axis_name="c", subcore_axis_name="s",
                                  num_cores=NC, num_subcores=NS)

    @functools.partial(
        pl.kernel,
        out_type=jax.ShapeDtypeStruct((N_PAIR, D_H), jnp.float32),
        mesh=mesh,
        scratch_types=[
            pltpu.VMEM((PAIR_W,), jnp.int32),
            pltpu.VMEM((32, D_H), jnp.float32),
            pltpu.VMEM((32, D_H), jnp.float32),
            pltpu.VMEM((32,), jnp.int32),
            pltpu.VMEM((32,), jnp.int32),
            pltpu.SemaphoreType.DMA,
            pltpu.SemaphoreType.DMA,
            pltpu.SemaphoreType.DMA,
            pltpu.SemaphoreType.DMA,
        ],
        compiler_params=pltpu.CompilerParams(needs_layout_passes=False),
    )
    def k4(ys_hbm, pos_hbm, yp_hbm, posall, yr0, yr1, ix0, ix1,
           sg0, sg1, sw0, sw1):
        c = lax.axis_index("c")
        s = lax.axis_index("s")
        w_own = c * NS + s
        pltpu.sync_copy(pos_hbm.at[pl.ds(w_own * PAIR_W, PAIR_W)], posall)
        bufs = (yr0, yr1)
        idxs = (ix0, ix1)
        gsems = (sg0, sg1)
        wsems = (sw0, sw1)
        writes = [None] * 8
        for ch in range(8):
            b = ch % 2
            if ch >= 2:
                writes[ch - 2].wait()
            ii = idxs[b]
            ii[pl.ds(0, 16)] = posall[pl.ds(ch * 32, 16)]
            ii[pl.ds(16, 16)] = posall[pl.ds(ch * 32 + 16, 16)]
            pltpu.async_copy(ys_hbm.at[ii], bufs[b], gsems[b]).wait()
            writes[ch] = pltpu.async_copy(
                bufs[b], yp_hbm.at[pl.ds(w_own * PAIR_W + ch * 32, 32)],
                wsems[b])
        writes[6].wait()
        writes[7].wait()

    return k4(ys, pos)


def _k5_body(shared_ref, yp_ref, wp_ref, out_ref):
    w0 = wp_ref[...][:, 0:1]
    w1 = wp_ref[...][:, 1:2]
    yp = yp_ref[...]
    out_ref[...] = (shared_ref[...] + w0 * yp[:, :D_H] + w1 * yp[:, D_H:])


def _k5(shared, ypairs2, wpair):
    return pl.pallas_call(
        _k5_body,
        grid=(NT,),
        in_specs=[
            pl.BlockSpec((TBLK, D_H), lambda i: (i, 0)),
            pl.BlockSpec((TBLK, 2 * D_H), lambda i: (i, 0)),
            pl.BlockSpec((TBLK, 2), lambda i: (i, 0)),
        ],
        out_specs=pl.BlockSpec((TBLK, D_H), lambda i: (i, 0)),
        out_shape=jax.ShapeDtypeStruct((T_TOTAL, D_H), jnp.float32),
    )(shared, ypairs2, wpair)


def kernel(x, Wg_router, W_gate, W_up, W_down, Ws_gate, Ws_up, Ws_down):
    B, S, H = x.shape
    x_flat = x.reshape(-1, H)
    epair, wpair, hist = _k0(x_flat, Wg_router)
    pos, xs, bmap, xsblk, bvalid = _k2(epair.reshape(-1), x_flat,
                                       hist.reshape(NW, N_E))
    ys = _k3(xs, W_gate, W_up, W_down, xsblk, bmap, bvalid)
    ypairs = _k4sc(ys, pos)
    shared = _k1s(x_flat, Ws_gate, Ws_up, Ws_down)
    out = _k5(shared, ypairs.reshape(T_TOTAL, 2 * D_H), wpair)
    return out.reshape(B, S, H)
